# baseline (speedup 1.0000x reference)
import functools

import jax
import jax.numpy as jnp
from jax.experimental import pallas as pl
from jax.experimental.pallas import tpu as pltpu
from jax.experimental.pallas import tpu_sc as plsc

EPS = 1e-8
NORMAL_T = 0.99
F = 8192
V = 4096
KSEL = 51
BLK = 256
NBLK = F // BLK
PREP_COLS = 1024
KEY_SHIFT = 12
KEY_BITS = 31 - KEY_SHIFT
SROWS = 32


_SC_NC, _SC_NS = 2, 16
_NW = _SC_NC * _SC_NS
_GB = 3 * F
_BPW = _GB // _NW
_VD = 128
_NCHUNK = 4
_CHUNK = _BPW // _NCHUNK


def _sc_gather(table, idx):
    mesh = plsc.VectorSubcoreMesh(core_axis_name="c", subcore_axis_name="s")

    @functools.partial(
        pl.kernel, mesh=mesh,
        out_type=jax.ShapeDtypeStruct((_GB, _VD), jnp.float32),
        scratch_types=[
            pltpu.VMEM((_CHUNK,), jnp.int32),
            pltpu.VMEM((_CHUNK, _VD), jnp.float32),
            pltpu.SemaphoreType.DMA,
        ],
    )
    def k(table_hbm, idx_hbm, out_hbm, idx_v, rows_v, sem):
        wid = jax.lax.axis_index("s") * _SC_NC + jax.lax.axis_index("c")
        for ch in range(_NCHUNK):
            base = wid * _BPW + ch * _CHUNK
            pltpu.sync_copy(idx_hbm.at[pl.ds(base, _CHUNK)], idx_v)
            pltpu.async_copy(table_hbm.at[idx_v], rows_v, sem).wait()
            pltpu.sync_copy(rows_v, out_hbm.at[pl.ds(base, _CHUNK)])

    return k(table, idx)


def _faceprep_kernel(v0_ref, v1_ref, v2_ref, facesT_ref, out_ref):
    v0 = v0_ref[...]
    v1 = v1_ref[...]
    v2 = v2_ref[...]
    e1 = v1 - v0
    e2 = v2 - v0

    def row(a, r):
        return a[r:r + 1, :]

    rnx = row(e1, 1) * row(e2, 2) - row(e1, 2) * row(e2, 1)
    rny = row(e1, 2) * row(e2, 0) - row(e1, 0) * row(e2, 2)
    rnz = row(e1, 0) * row(e2, 1) - row(e1, 1) * row(e2, 0)
    norm = jnp.sqrt(rnx * rnx + rny * rny + rnz * rnz) + EPS
    cent = (v0 + v1 + v2) / 3.0
    cx, cy, cz = row(cent, 0), row(cent, 1), row(cent, 2)
    c2 = cx * cx + cy * cy + cz * cz
    f0 = facesT_ref[0:1, :].astype(jnp.float32)
    f1 = facesT_ref[1:2, :].astype(jnp.float32)
    f2 = facesT_ref[2:3, :].astype(jnp.float32)
    w1 = jnp.where(f1 == f0, 0.0, 1.0)
    w2 = jnp.where((f2 == f0) | (f2 == f1), 0.0, 1.0)
    neg_w0n = -(row(v0, 0) * rnx + row(v0, 1) * rny + row(v0, 2) * rnz)
    ones_r = jnp.ones((1, PREP_COLS), jnp.float32)

    out_ref[0:3, :] = v0
    out_ref[3:4, :] = ones_r
    out_ref[4:7, :] = v1
    out_ref[7:8, :] = ones_r
    out_ref[8:11, :] = v2
    out_ref[11:12, :] = ones_r
    out_ref[12:13, :] = rnx
    out_ref[13:14, :] = rny
    out_ref[14:15, :] = rnz
    out_ref[15:16, :] = neg_w0n
    out_ref[16:17, :] = rnx / norm
    out_ref[17:18, :] = rny / norm
    out_ref[18:19, :] = rnz / norm
    out_ref[19:22, :] = cent
    out_ref[22:23, :] = c2
    out_ref[23:24, :] = f0
    out_ref[24:25, :] = f1
    out_ref[25:26, :] = f2
    out_ref[26:27, :] = w1
    out_ref[27:28, :] = w2
    out_ref[28:SROWS, :] = jnp.zeros((SROWS - 28, PREP_COLS), jnp.float32)


def _main_kernel(soa_ref, soaT_ref, p_ref, out_ref):
    s = soa_ref[...]
    st = soaT_ref[...]

    def jrow(r):
        return s[r:r + 1, :]

    def icol(r):
        return st[:, r:r + 1]

    def mm(lhs, rhs, prec=jax.lax.Precision.DEFAULT):
        return jax.lax.dot_general(
            lhs, rhs, (((1,), (0,)), ((), ())), precision=prec)

    cdot = mm(st[:, 19:22], s[19:22, :])
    d2 = icol(22) + jrow(22) - 2.0 * cdot
    key = jax.lax.shift_right_logical(
        jax.lax.bitcast_convert_type(jnp.maximum(d2, 0.0), jnp.int32),
        KEY_SHIFT)
    col = jax.lax.broadcasted_iota(jnp.int32, (BLK, F), 1)
    ones_v = jnp.ones((F, 1), jnp.float32)

    def body_k(_, lh):
        lo, hi = lh
        mid = lo + ((hi - lo) >> 1)
        cnt = mm(jnp.where(key <= mid, 1.0, 0.0), ones_v)
        geq = cnt >= float(KSEL)
        return (jnp.where(geq, lo, mid + 1), jnp.where(geq, mid, hi))

    lo0 = jnp.zeros((BLK, 1), jnp.int32)
    hi0 = jnp.full((BLK, 1), (1 << KEY_BITS) - 1, jnp.int32)
    kstar, _ = jax.lax.fori_loop(0, KEY_BITS, body_k, (lo0, hi0))

    k0 = jnp.min(key, axis=1, keepdims=True)
    c0 = jnp.min(jnp.where(key == k0, col, F), axis=1, keepdims=True)
    sel = (key <= kstar) & jnp.logical_not((key == k0) & (col == c0))

    v0ni = icol(0) * icol(12) + icol(1) * icol(13) + icol(2) * icol(14)
    lhs_d = jnp.concatenate([st[:, 12:15], -v0ni], axis=1)
    da = mm(lhs_d, s[0:4, :])
    db = mm(lhs_d, s[4:8, :])
    dc = mm(lhs_d, s[8:12, :])
    m1 = jnp.minimum(jnp.minimum(da * db, da * dc), db * dc)
    rhs_e = s[12:16, :]
    ea = mm(st[:, 0:4], rhs_e)
    eb = mm(st[:, 4:8], rhs_e)
    ec = mm(st[:, 8:12], rhs_e)
    m2 = jnp.minimum(jnp.minimum(ea * eb, ea * ec), eb * ec)
    noncop = jnp.maximum(m1, m2) <= 0
    ndot = jnp.abs(mm(st[:, 16:19], s[16:19, :]))
    not_coplanar = ndot <= NORMAL_T

    fi0, fi1, fi2 = icol(23), icol(24), icol(25)
    fj0, fj1, fj2 = jrow(23), jrow(24), jrow(25)
    pres0 = (fi0 == fj0) | (fi0 == fj1) | (fi0 == fj2)
    pres1 = (fi1 == fj0) | (fi1 == fj1) | (fi1 == fj2)
    pres2 = (fi2 == fj0) | (fi2 == fj1) | (fi2 == fj2)
    shared = jnp.where(pres0, 1.0, 0.0) + \
        jnp.where(pres1, icol(26), 0.0) + \
        jnp.where(pres2, icol(27), 0.0)
    not_adjacent = shared < 2.0

    collision = noncop & not_coplanar & not_adjacent & sel
    pcol = p_ref[0]
    partial = jnp.sum(jnp.where(collision, pcol, 0.0))
    out_ref[...] = jnp.full((1, 1, 128), partial, jnp.float32)


def kernel(vertices, faces, face_probabilities):
    facesT = faces.astype(jnp.int32).T
    vpad = jnp.pad(vertices.astype(jnp.float32), ((0, 0), (0, _VD - 3)))
    gathered = _sc_gather(vpad, facesT.reshape(_GB))
    gt = gathered[:, 0:3].T
    v0g = gt[:, 0:F]
    v1g = gt[:, F:2 * F]
    v2g = gt[:, 2 * F:3 * F]
    soa = pl.pallas_call(
        _faceprep_kernel,
        grid=(F // PREP_COLS,),
        in_specs=[
            pl.BlockSpec((3, PREP_COLS), lambda b: (0, b)),
            pl.BlockSpec((3, PREP_COLS), lambda b: (0, b)),
            pl.BlockSpec((3, PREP_COLS), lambda b: (0, b)),
            pl.BlockSpec((3, PREP_COLS), lambda b: (0, b)),
        ],
        out_specs=pl.BlockSpec((SROWS, PREP_COLS), lambda b: (0, b)),
        out_shape=jax.ShapeDtypeStruct((SROWS, F), jnp.float32),
        compiler_params=pltpu.CompilerParams(
            dimension_semantics=("parallel",)),
    )(v0g, v1g, v2g, facesT)
    soaT = soa.T
    p3 = face_probabilities.reshape(NBLK, BLK, 1)
    out = pl.pallas_call(
        _main_kernel,
        grid=(NBLK,),
        in_specs=[
            pl.BlockSpec((SROWS, F), lambda b: (0, 0)),
            pl.BlockSpec((BLK, SROWS), lambda b: (b, 0)),
            pl.BlockSpec((1, BLK, 1), lambda b: (b, 0, 0)),
        ],
        out_specs=pl.BlockSpec((1, 1, 128), lambda b: (b, 0, 0)),
        out_shape=jax.ShapeDtypeStruct((NBLK, 1, 128), jnp.float32),
        compiler_params=pltpu.CompilerParams(
            dimension_semantics=("parallel",)),
    )(soa, soaT, p3)
    return jnp.sum(out[:, 0, 0])


# speedup vs baseline: 1.1314x; 1.1314x over previous
import functools

import jax
import jax.numpy as jnp
from jax.experimental import pallas as pl
from jax.experimental.pallas import tpu as pltpu
from jax.experimental.pallas import tpu_sc as plsc

EPS = 1e-8
NORMAL_T = 0.99
F = 8192
V = 4096
KSEL = 51
BLK = 256
NBLK = F // BLK
PREP_COLS = 1024
KEY_SHIFT = 12
KEY_BITS = 31 - KEY_SHIFT
SROWS = 32


_SC_NC, _SC_NS = 2, 16
_NW = _SC_NC * _SC_NS
_GB = 3 * F
_BPW = _GB // _NW
_VD = 128
_NCHUNK = 4
_CHUNK = _BPW // _NCHUNK


def _sc_gather(table, idx):
    mesh = plsc.VectorSubcoreMesh(core_axis_name="c", subcore_axis_name="s")

    @functools.partial(
        pl.kernel, mesh=mesh,
        out_type=jax.ShapeDtypeStruct((_GB, _VD), jnp.float32),
        scratch_types=[
            pltpu.VMEM((_CHUNK,), jnp.int32),
            pltpu.VMEM((_CHUNK, _VD), jnp.float32),
            pltpu.SemaphoreType.DMA,
        ],
    )
    def k(table_hbm, idx_hbm, out_hbm, idx_v, rows_v, sem):
        wid = jax.lax.axis_index("s") * _SC_NC + jax.lax.axis_index("c")
        for ch in range(_NCHUNK):
            base = wid * _BPW + ch * _CHUNK
            pltpu.sync_copy(idx_hbm.at[pl.ds(base, _CHUNK)], idx_v)
            pltpu.async_copy(table_hbm.at[idx_v], rows_v, sem).wait()
            pltpu.sync_copy(rows_v, out_hbm.at[pl.ds(base, _CHUNK)])

    return k(table, idx)


def _faceprep_kernel(v0_ref, v1_ref, v2_ref, facesT_ref, out_ref):
    v0 = v0_ref[...]
    v1 = v1_ref[...]
    v2 = v2_ref[...]
    e1 = v1 - v0
    e2 = v2 - v0

    def row(a, r):
        return a[r:r + 1, :]

    rnx = row(e1, 1) * row(e2, 2) - row(e1, 2) * row(e2, 1)
    rny = row(e1, 2) * row(e2, 0) - row(e1, 0) * row(e2, 2)
    rnz = row(e1, 0) * row(e2, 1) - row(e1, 1) * row(e2, 0)
    norm = jnp.sqrt(rnx * rnx + rny * rny + rnz * rnz) + EPS
    cent = (v0 + v1 + v2) / 3.0
    cx, cy, cz = row(cent, 0), row(cent, 1), row(cent, 2)
    c2 = cx * cx + cy * cy + cz * cz
    f0 = facesT_ref[0:1, :].astype(jnp.float32)
    f1 = facesT_ref[1:2, :].astype(jnp.float32)
    f2 = facesT_ref[2:3, :].astype(jnp.float32)
    w1 = jnp.where(f1 == f0, 0.0, 1.0)
    w2 = jnp.where((f2 == f0) | (f2 == f1), 0.0, 1.0)
    neg_w0n = -(row(v0, 0) * rnx + row(v0, 1) * rny + row(v0, 2) * rnz)
    ones_r = jnp.ones((1, PREP_COLS), jnp.float32)

    out_ref[0:3, :] = v0
    out_ref[3:4, :] = ones_r
    out_ref[4:7, :] = v1
    out_ref[7:8, :] = ones_r
    out_ref[8:11, :] = v2
    out_ref[11:12, :] = ones_r
    out_ref[12:13, :] = rnx
    out_ref[13:14, :] = rny
    out_ref[14:15, :] = rnz
    out_ref[15:16, :] = neg_w0n
    out_ref[16:17, :] = rnx / norm
    out_ref[17:18, :] = rny / norm
    out_ref[18:19, :] = rnz / norm
    out_ref[19:22, :] = cent
    out_ref[22:23, :] = c2
    out_ref[23:24, :] = f0
    out_ref[24:25, :] = f1
    out_ref[25:26, :] = f2
    out_ref[26:27, :] = w1
    out_ref[27:28, :] = w2
    out_ref[28:SROWS, :] = jnp.zeros((SROWS - 28, PREP_COLS), jnp.float32)


def _main_kernel(soa_ref, soaT_ref, p_ref, out_ref):
    s = soa_ref[...]
    st = soaT_ref[...]

    def jrow(r):
        return s[r:r + 1, :]

    def icol(r):
        return st[:, r:r + 1]

    def mm(lhs, rhs, prec=jax.lax.Precision.DEFAULT):
        return jax.lax.dot_general(
            lhs, rhs, (((1,), (0,)), ((), ())), precision=prec)

    cdot = mm(st[:, 19:22], s[19:22, :])
    d2 = icol(22) + jrow(22) - 2.0 * cdot
    key = jax.lax.shift_right_logical(
        jax.lax.bitcast_convert_type(jnp.maximum(d2, 0.0), jnp.int32),
        KEY_SHIFT)
    col = jax.lax.broadcasted_iota(jnp.int32, (BLK, F), 1)
    ones_v = jnp.ones((F, 1), jnp.float32)

    def body_k(_, lh):
        lo, hi = lh
        mid = lo + ((hi - lo) >> 1)
        cnt = jnp.sum(jnp.where(key <= mid, 1, 0), axis=1, keepdims=True)
        geq = cnt >= KSEL
        return (jnp.where(geq, lo, mid + 1), jnp.where(geq, mid, hi))

    lo0 = jnp.zeros((BLK, 1), jnp.int32)
    hi0 = jnp.full((BLK, 1), (1 << KEY_BITS) - 1, jnp.int32)
    kstar, _ = jax.lax.fori_loop(0, KEY_BITS, body_k, (lo0, hi0))

    k0 = jnp.min(key, axis=1, keepdims=True)
    c0 = jnp.min(jnp.where(key == k0, col, F), axis=1, keepdims=True)
    sel = (key <= kstar) & jnp.logical_not((key == k0) & (col == c0))

    v0ni = icol(0) * icol(12) + icol(1) * icol(13) + icol(2) * icol(14)
    lhs_d = jnp.concatenate([st[:, 12:15], -v0ni], axis=1)
    da = mm(lhs_d, s[0:4, :])
    db = mm(lhs_d, s[4:8, :])
    dc = mm(lhs_d, s[8:12, :])
    m1 = jnp.minimum(jnp.minimum(da * db, da * dc), db * dc)
    rhs_e = s[12:16, :]
    ea = mm(st[:, 0:4], rhs_e)
    eb = mm(st[:, 4:8], rhs_e)
    ec = mm(st[:, 8:12], rhs_e)
    m2 = jnp.minimum(jnp.minimum(ea * eb, ea * ec), eb * ec)
    noncop = jnp.maximum(m1, m2) <= 0
    ndot = jnp.abs(mm(st[:, 16:19], s[16:19, :]))
    not_coplanar = ndot <= NORMAL_T

    fi0, fi1, fi2 = icol(23), icol(24), icol(25)
    fj0, fj1, fj2 = jrow(23), jrow(24), jrow(25)
    pres0 = (fi0 == fj0) | (fi0 == fj1) | (fi0 == fj2)
    pres1 = (fi1 == fj0) | (fi1 == fj1) | (fi1 == fj2)
    pres2 = (fi2 == fj0) | (fi2 == fj1) | (fi2 == fj2)
    shared = jnp.where(pres0, 1.0, 0.0) + \
        jnp.where(pres1, icol(26), 0.0) + \
        jnp.where(pres2, icol(27), 0.0)
    not_adjacent = shared < 2.0

    collision = noncop & not_coplanar & not_adjacent & sel
    pcol = p_ref[0]
    partial = jnp.sum(jnp.where(collision, pcol, 0.0))
    out_ref[...] = jnp.full((1, 1, 128), partial, jnp.float32)


def kernel(vertices, faces, face_probabilities):
    facesT = faces.astype(jnp.int32).T
    vpad = jnp.pad(vertices.astype(jnp.float32), ((0, 0), (0, _VD - 3)))
    gathered = _sc_gather(vpad, facesT.reshape(_GB))
    gt = gathered[:, 0:3].T
    v0g = gt[:, 0:F]
    v1g = gt[:, F:2 * F]
    v2g = gt[:, 2 * F:3 * F]
    soa = pl.pallas_call(
        _faceprep_kernel,
        grid=(F // PREP_COLS,),
        in_specs=[
            pl.BlockSpec((3, PREP_COLS), lambda b: (0, b)),
            pl.BlockSpec((3, PREP_COLS), lambda b: (0, b)),
            pl.BlockSpec((3, PREP_COLS), lambda b: (0, b)),
            pl.BlockSpec((3, PREP_COLS), lambda b: (0, b)),
        ],
        out_specs=pl.BlockSpec((SROWS, PREP_COLS), lambda b: (0, b)),
        out_shape=jax.ShapeDtypeStruct((SROWS, F), jnp.float32),
        compiler_params=pltpu.CompilerParams(
            dimension_semantics=("parallel",)),
    )(v0g, v1g, v2g, facesT)
    soaT = soa.T
    p3 = face_probabilities.reshape(NBLK, BLK, 1)
    out = pl.pallas_call(
        _main_kernel,
        grid=(NBLK,),
        in_specs=[
            pl.BlockSpec((SROWS, F), lambda b: (0, 0)),
            pl.BlockSpec((BLK, SROWS), lambda b: (b, 0)),
            pl.BlockSpec((1, BLK, 1), lambda b: (b, 0, 0)),
        ],
        out_specs=pl.BlockSpec((1, 1, 128), lambda b: (b, 0, 0)),
        out_shape=jax.ShapeDtypeStruct((NBLK, 1, 128), jnp.float32),
        compiler_params=pltpu.CompilerParams(
            dimension_semantics=("parallel",)),
    )(soa, soaT, p3)
    return jnp.sum(out[:, 0, 0])
